# baseline (speedup 1.0000x reference)
import functools

import jax
import jax.numpy as jnp
from jax import lax
from jax.experimental import pallas as pl
from jax.experimental.pallas import tpu as pltpu
from jax.experimental.pallas import tpu_sc as plsc

N = 10000
D = 128
NEG = 32

NW = 32
ROWS_PER_W = 320
NPAD = NW * ROWS_PER_W
GROUP = 4
N_GROUPS = ROWS_PER_W // GROUP
BLK = 16
GROUPS_PER_BLK = BLK // GROUP
N_BLKS = ROWS_PER_W // BLK


def _tc_normalize_body(z1_ref, z2_ref, o1_ref, o2_ref, h1_ref, h2_ref):
    x1 = z1_ref[...]
    x2 = z2_ref[...]
    n1 = jnp.maximum(jnp.sqrt(jnp.sum(x1 * x1, axis=1, keepdims=True)), 1e-12)
    n2 = jnp.maximum(jnp.sqrt(jnp.sum(x2 * x2, axis=1, keepdims=True)), 1e-12)
    y1 = x1 / n1
    y2 = x2 / n2
    o1_ref[...] = y1
    o2_ref[...] = y2
    h1_ref[...] = y1.astype(jnp.bfloat16)
    h2_ref[...] = y2.astype(jnp.bfloat16)


def _tc_normalize(z1, z2):
    blk = 1000
    grid = (N // blk,)
    spec = pl.BlockSpec((blk, D), lambda i: (i, 0))
    return pl.pallas_call(
        _tc_normalize_body,
        grid=grid,
        in_specs=[spec, spec],
        out_specs=[spec, spec, spec, spec],
        out_shape=[
            jax.ShapeDtypeStruct((NPAD, D), jnp.float32),
            jax.ShapeDtypeStruct((NPAD, D), jnp.float32),
            jax.ShapeDtypeStruct((NPAD, D), jnp.bfloat16),
            jax.ShapeDtypeStruct((NPAD, D), jnp.bfloat16),
        ],
    )(z1, z2)


def _sc_score_body(z1n_hbm, z2n_hbm, z1h_hbm, z2h_hbm, neg_hbm, out_hbm,
                   idxb0, idxb1, z1b0, z1b1, z2b0, z2b1, z1h0, z1h1,
                   gbuf0, gbuf1, out_v, spm,
                   gsem0, gsem1, ssem0, ssem1, semspm):
    info = plsc.get_sparse_core_info()
    nc = info.num_cores
    sid = lax.axis_index("s")
    wid = sid * nc + lax.axis_index("c")
    base = wid * ROWS_PER_W

    stripe = NPAD // 16
    spm_cp = pltpu.make_async_copy(
        z2h_hbm.at[pl.ds(sid * stripe, stripe)],
        spm.at[pl.ds(sid * stripe, stripe)], semspm)
    spm_cp.start()

    idxbs = (idxb0, idxb1)
    z1bs = (z1b0, z1b1)
    z2bs = (z2b0, z2b1)
    z1hs = (z1h0, z1h1)
    ssems = (ssem0, ssem1)
    gbufs = (gbuf0, gbuf1)
    gsems = (gsem0, gsem1)

    def stage_copies(blk, pb):
        idx_off = jnp.minimum(wid * N_GROUPS + blk * GROUPS_PER_BLK,
                              N * NEG // 128 - GROUPS_PER_BLK)
        row_off = jnp.minimum(base + blk * BLK, N - BLK)
        return (
            pltpu.make_async_copy(
                neg_hbm.at[pl.ds(idx_off, GROUPS_PER_BLK)],
                idxbs[pb], ssems[pb]),
            pltpu.make_async_copy(
                z1n_hbm.at[pl.ds(row_off, BLK)], z1bs[pb], ssems[pb]),
            pltpu.make_async_copy(
                z2n_hbm.at[pl.ds(row_off, BLK)], z2bs[pb], ssems[pb]),
            pltpu.make_async_copy(
                z1h_hbm.at[pl.ds(row_off, BLK)], z1hs[pb], ssems[pb]),
        )

    def stage_start(blk, pb):
        for cp in stage_copies(blk, pb):
            cp.start()

    def stage_wait(blk, pb):
        for cp in stage_copies(blk, pb):
            cp.wait()

    def gather_start(pb, gsub, bg):
        pltpu.make_async_copy(
            spm.at[idxbs[pb].at[gsub]], gbufs[bg], gsems[bg]).start()

    def gather_wait(pb, gsub, bg):
        pltpu.make_async_copy(
            spm.at[idxbs[pb].at[gsub]], gbufs[bg], gsems[bg]).wait()

    stage_start(0, 0)
    stage_start(1, 1)
    spm_cp.wait()
    plsc.subcore_barrier()
    stage_wait(0, 0)
    gather_start(0, 0, 0)
    gather_start(0, 1, 1)

    lanes = jnp.arange(16, dtype=jnp.int32)
    inv_neg = 1.0 / NEG

    dnums = lax.GatherDimensionNumbers(
        offset_dims=(), collapsed_slice_dims=(0,), start_index_map=(0,))

    def lane_sum(v):
        for sh in (8, 4, 2, 1):
            perm = (lanes ^ sh)[:, None]
            v = v + lax.gather(
                v, perm, dimension_numbers=dnums, slice_sizes=(1,),
                mode=lax.GatherScatterMode.PROMISE_IN_BOUNDS)
        return v

    def widen_pair(ref, row, q):
        w = ref[row, pl.ds(q * 16, 16)]
        even = plsc.bitcast(jnp.left_shift(w, 16), jnp.float32)
        odd = plsc.bitcast(w, jnp.float32)
        return even, odd

    def do_block(B, pb):
        res = jnp.zeros((16,), jnp.float32)
        for gsub in range(GROUPS_PER_BLK):
            bg = gsub % 2
            gather_wait(pb, gsub, bg)
            gb = gbufs[bg]
            for r in range(GROUP):
                lrow = gsub * GROUP + r
                z1s = [widen_pair(z1hs[pb], lrow, q) for q in range(4)]

                def jc_body(jc, accs):
                    jb = jc * 8
                    new = list(accs)
                    for jj in range(8):
                        for q in range(4):
                            ge, go = widen_pair(gb, r * NEG + jb + jj, q)
                            new[2 * q] = new[2 * q] + z1s[q][0] * ge
                            new[2 * q + 1] = new[2 * q + 1] + z1s[q][1] * go
                    return tuple(new)
                accs = lax.fori_loop(
                    0, NEG // 8, jc_body,
                    tuple(jnp.zeros((16,), jnp.float32) for _ in range(8)))
                negacc = (((accs[0] + accs[1]) + (accs[2] + accs[3]))
                          + ((accs[4] + accs[5]) + (accs[6] + accs[7])))

                posacc = jnp.zeros((16,), jnp.float32)
                for d in range(8):
                    posacc = posacc + z1bs[pb][lrow, pl.ds(d * 16, 16)] * \
                        z2bs[pb][lrow, pl.ds(d * 16, 16)]

                s = lane_sum(posacc - inv_neg * negacc)
                res = jnp.where(lanes == lrow, s, res)

            if gsub < 2:
                gather_start(pb, gsub + 2, bg)
            else:
                if gsub == 2:
                    @pl.when(B + 1 < N_BLKS)
                    def _():
                        stage_wait(B + 1, 1 - pb)

                @pl.when(B + 1 < N_BLKS)
                def _():
                    gather_start(1 - pb, gsub - 2, bg)
        out_v[pl.ds(B * BLK, BLK)] = res
        @pl.when(B + 2 < N_BLKS)
        def _():
            stage_start(B + 2, pb)

    def pair_body(k, carry):
        do_block(2 * k, 0)
        do_block(2 * k + 1, 1)
        return carry

    lax.fori_loop(0, N_BLKS // 2, pair_body, 0)
    pltpu.sync_copy(out_v, out_hbm.at[pl.ds(base, ROWS_PER_W)])


def _sc_score(z1n, z2n, z1h, z2h, neg_r):
    mesh = plsc.VectorSubcoreMesh(core_axis_name="c", subcore_axis_name="s")
    kfn = functools.partial(
        pl.kernel,
        mesh=mesh,
        compiler_params=pltpu.CompilerParams(needs_layout_passes=False),
        out_type=jax.ShapeDtypeStruct((NPAD,), jnp.float32),
        scratch_types=[
            pltpu.VMEM((GROUPS_PER_BLK, 128), jnp.int32),
            pltpu.VMEM((GROUPS_PER_BLK, 128), jnp.int32),
            pltpu.VMEM((BLK, D), jnp.float32),
            pltpu.VMEM((BLK, D), jnp.float32),
            pltpu.VMEM((BLK, D), jnp.float32),
            pltpu.VMEM((BLK, D), jnp.float32),
            pltpu.VMEM((BLK, D // 2), jnp.int32),
            pltpu.VMEM((BLK, D // 2), jnp.int32),
            pltpu.VMEM((GROUP * NEG, D // 2), jnp.int32),
            pltpu.VMEM((GROUP * NEG, D // 2), jnp.int32),
            pltpu.VMEM((ROWS_PER_W,), jnp.float32),
            pltpu.VMEM_SHARED((NPAD, D // 2), jnp.int32),
            pltpu.SemaphoreType.DMA,
            pltpu.SemaphoreType.DMA,
            pltpu.SemaphoreType.DMA,
            pltpu.SemaphoreType.DMA,
            pltpu.SemaphoreType.DMA,
        ],
    )(_sc_score_body)
    return kfn(z1n, z2n, z1h, z2h, neg_r)


def kernel(z1, z2, negative_samples):
    neg_r = negative_samples.astype(jnp.int32).reshape(N * NEG // 128, 128)
    z1n, z2n, z1h, z2h = _tc_normalize(z1, z2)
    z1p = lax.bitcast_convert_type(z1h.reshape(NPAD, D // 2, 2), jnp.int32)
    z2p = lax.bitcast_convert_type(z2h.reshape(NPAD, D // 2, 2), jnp.int32)
    out = _sc_score(z1n, z2n, z1p, z2p, neg_r)
    return out[:N]


# speedup vs baseline: 1.4833x; 1.4833x over previous
import functools

import jax
import jax.numpy as jnp
from jax import lax
from jax.experimental import pallas as pl
from jax.experimental.pallas import tpu as pltpu
from jax.experimental.pallas import tpu_sc as plsc

N = 10000
D = 128
NEG = 32

NW = 32
ROWS_PER_W = 320
NPAD = NW * ROWS_PER_W
GROUP = 4
N_GROUPS = ROWS_PER_W // GROUP
BLK = 16
GROUPS_PER_BLK = BLK // GROUP
N_BLKS = ROWS_PER_W // BLK


def _tc_normalize_body(z1_ref, z2_ref, o1_ref, o2_ref):
    x1 = z1_ref[...]
    x2 = z2_ref[...]
    n1 = jnp.maximum(jnp.sqrt(jnp.sum(x1 * x1, axis=1, keepdims=True)), 1e-12)
    n2 = jnp.maximum(jnp.sqrt(jnp.sum(x2 * x2, axis=1, keepdims=True)), 1e-12)
    o1_ref[...] = x1 / n1
    o2_ref[...] = x2 / n2


def _tc_normalize(z1, z2):
    blk = 1000
    grid = (N // blk,)
    spec = pl.BlockSpec((blk, D), lambda i: (i, 0))
    return pl.pallas_call(
        _tc_normalize_body,
        grid=grid,
        in_specs=[spec, spec],
        out_specs=[spec, spec],
        out_shape=[
            jax.ShapeDtypeStruct((NPAD, D), jnp.float32),
            jax.ShapeDtypeStruct((NPAD, D), jnp.float32),
        ],
    )(z1, z2)


def _sc_score_body(z1n_hbm, z2n_hbm, neg_hbm, out_hbm,
                   idxb0, idxb1, z1b0, z1b1, z2b0, z2b1,
                   gbuf0, gbuf1, out_v, spm,
                   gsem0, gsem1, ssem0, ssem1, semspm):
    info = plsc.get_sparse_core_info()
    nc = info.num_cores
    sid = lax.axis_index("s")
    wid = sid * nc + lax.axis_index("c")
    base = wid * ROWS_PER_W

    stripe = NPAD // 16
    spm_cp = pltpu.make_async_copy(
        z2n_hbm.at[pl.ds(sid * stripe, stripe)],
        spm.at[pl.ds(sid * stripe, stripe)], semspm)
    spm_cp.start()

    idxbs = (idxb0, idxb1)
    z1bs = (z1b0, z1b1)
    z2bs = (z2b0, z2b1)
    ssems = (ssem0, ssem1)
    gbufs = (gbuf0, gbuf1)
    gsems = (gsem0, gsem1)

    def stage_copies(blk, pb):
        idx_off = jnp.minimum(wid * N_GROUPS + blk * GROUPS_PER_BLK,
                              N * NEG // 128 - GROUPS_PER_BLK)
        row_off = jnp.minimum(base + blk * BLK, N - BLK)
        return (
            pltpu.make_async_copy(
                neg_hbm.at[pl.ds(idx_off, GROUPS_PER_BLK)],
                idxbs[pb], ssems[pb]),
            pltpu.make_async_copy(
                z1n_hbm.at[pl.ds(row_off, BLK)], z1bs[pb], ssems[pb]),
            pltpu.make_async_copy(
                z2n_hbm.at[pl.ds(row_off, BLK)], z2bs[pb], ssems[pb]),
        )

    def stage_start(blk, pb):
        for cp in stage_copies(blk, pb):
            cp.start()

    def stage_wait(blk, pb):
        for cp in stage_copies(blk, pb):
            cp.wait()

    def gather_start(pb, gsub, bg):
        pltpu.make_async_copy(
            spm.at[idxbs[pb].at[gsub]], gbufs[bg], gsems[bg]).start()

    def gather_wait(pb, gsub, bg):
        pltpu.make_async_copy(
            spm.at[idxbs[pb].at[gsub]], gbufs[bg], gsems[bg]).wait()

    stage_start(0, 0)
    stage_start(1, 1)
    spm_cp.wait()
    plsc.subcore_barrier()
    stage_wait(0, 0)
    gather_start(0, 0, 0)
    gather_start(0, 1, 1)

    lanes = jnp.arange(16, dtype=jnp.int32)
    inv_neg = 1.0 / NEG

    dnums = lax.GatherDimensionNumbers(
        offset_dims=(), collapsed_slice_dims=(0,), start_index_map=(0,))

    def lane_sum(v):
        for sh in (8, 4, 2, 1):
            perm = (lanes ^ sh)[:, None]
            v = v + lax.gather(
                v, perm, dimension_numbers=dnums, slice_sizes=(1,),
                mode=lax.GatherScatterMode.PROMISE_IN_BOUNDS)
        return v

    def do_block(B, pb):
        res = jnp.zeros((16,), jnp.float32)
        for gsub in range(GROUPS_PER_BLK):
            bg = gsub % 2
            gather_wait(pb, gsub, bg)
            gb = gbufs[bg]
            for r in range(GROUP):
                lrow = gsub * GROUP + r
                z1r = [z1bs[pb][lrow, pl.ds(d * 16, 16)] for d in range(8)]

                def jc_body(jc, accs):
                    jb = jc * 8
                    new = list(accs)
                    for jj in range(8):
                        for d in range(8):
                            new[d] = new[d] + z1r[d] * gb[
                                r * NEG + jb + jj, pl.ds(d * 16, 16)]
                    return tuple(new)
                accs = lax.fori_loop(
                    0, NEG // 8, jc_body,
                    tuple(jnp.zeros((16,), jnp.float32) for _ in range(8)))
                negacc = (((accs[0] + accs[1]) + (accs[2] + accs[3]))
                          + ((accs[4] + accs[5]) + (accs[6] + accs[7])))

                posacc = jnp.zeros((16,), jnp.float32)
                for d in range(8):
                    posacc = posacc + z1r[d] * z2bs[pb][
                        lrow, pl.ds(d * 16, 16)]

                s = lane_sum(posacc - inv_neg * negacc)
                res = jnp.where(lanes == lrow, s, res)

            if gsub < 2:
                gather_start(pb, gsub + 2, bg)
            else:
                if gsub == 2:
                    @pl.when(B + 1 < N_BLKS)
                    def _():
                        stage_wait(B + 1, 1 - pb)

                @pl.when(B + 1 < N_BLKS)
                def _():
                    gather_start(1 - pb, gsub - 2, bg)
        out_v[pl.ds(B * BLK, BLK)] = res
        @pl.when(B + 2 < N_BLKS)
        def _():
            stage_start(B + 2, pb)

    def pair_body(k, carry):
        do_block(2 * k, 0)
        do_block(2 * k + 1, 1)
        return carry

    lax.fori_loop(0, N_BLKS // 2, pair_body, 0)
    pltpu.sync_copy(out_v, out_hbm.at[pl.ds(base, ROWS_PER_W)])


def _sc_score(z1n, z2n, neg_r):
    mesh = plsc.VectorSubcoreMesh(core_axis_name="c", subcore_axis_name="s")
    kfn = functools.partial(
        pl.kernel,
        mesh=mesh,
        out_type=jax.ShapeDtypeStruct((NPAD,), jnp.float32),
        scratch_types=[
            pltpu.VMEM((GROUPS_PER_BLK, 128), jnp.int32),
            pltpu.VMEM((GROUPS_PER_BLK, 128), jnp.int32),
            pltpu.VMEM((BLK, D), jnp.float32),
            pltpu.VMEM((BLK, D), jnp.float32),
            pltpu.VMEM((BLK, D), jnp.float32),
            pltpu.VMEM((BLK, D), jnp.float32),
            pltpu.VMEM((GROUP * NEG, D), jnp.float32),
            pltpu.VMEM((GROUP * NEG, D), jnp.float32),
            pltpu.VMEM((ROWS_PER_W,), jnp.float32),
            pltpu.VMEM_SHARED((NPAD, D), jnp.float32),
            pltpu.SemaphoreType.DMA,
            pltpu.SemaphoreType.DMA,
            pltpu.SemaphoreType.DMA,
            pltpu.SemaphoreType.DMA,
            pltpu.SemaphoreType.DMA,
        ],
    )(_sc_score_body)
    return kfn(z1n, z2n, neg_r)


def kernel(z1, z2, negative_samples):
    neg_r = negative_samples.astype(jnp.int32).reshape(N * NEG // 128, 128)
    z1n, z2n = _tc_normalize(z1, z2)
    out = _sc_score(z1n, z2n, neg_r)
    return out[:N]
